# baseline (speedup 1.0000x reference)
import functools

import jax
import jax.numpy as jnp
from jax import lax
from jax.experimental import pallas as pl
from jax.experimental.pallas import tpu as pltpu
from jax.experimental.pallas import tpu_sc as plsc

N = 10000
D = 128
E = 320000
OUTP = 3

NC = 2
NS = 16
NW = NC * NS
IR = 100
UR = 8
UNITS = E // (IR * UR)
CB = 96
RSPLIT = 632
RLAST = N - 15 * RSPLIT

_mesh = plsc.VectorSubcoreMesh(core_axis_name="c", subcore_axis_name="s")


def _seg_body(with_deg, h_hbm, ei_hbm, z2_hbm, z1_hbm,
              agg_hbm, deg0_hbm, deg1_hbm,
              src_v, dst_v, rows_a, rows_b, rows_c, ones_v, dbuf_v,
              acc_s, deg_s,
              sem_a, sem_b, sem_c, sem_d, sem_e, sem_f, isem_s, isem_d):
    c = lax.axis_index("c")
    s = lax.axis_index("s")
    w = c * NS + s

    def bounce_rows(src_at, dst_at, row0, nrows):
        full, rem = nrows // CB, nrows % CB
        for i in range(full):
            r = pl.multiple_of(row0 + i * CB, 8)
            pltpu.sync_copy(src_at(r, CB), rows_a.at[pl.ds(0, CB)])
            pltpu.sync_copy(rows_a.at[pl.ds(0, CB)], dst_at(r, CB))
        if rem:
            r = pl.multiple_of(row0 + full * CB, 8)
            pltpu.sync_copy(src_at(r, rem), rows_a.at[pl.ds(0, rem)])
            pltpu.sync_copy(rows_a.at[pl.ds(0, rem)], dst_at(r, rem))

    row0 = pl.multiple_of(s * RSPLIT, 8)

    @pl.when(s < NS - 1)
    def _():
        bounce_rows(lambda r, n: z2_hbm.at[pl.ds(r, n)],
                    lambda r, n: acc_s.at[pl.ds(r, n)], row0, RSPLIT)

    @pl.when(s == NS - 1)
    def _():
        bounce_rows(lambda r, n: z2_hbm.at[pl.ds(r, n)],
                    lambda r, n: acc_s.at[pl.ds(r, n)],
                    (NS - 1) * RSPLIT, RLAST)

    if with_deg:
        @pl.when(s < 10)
        def _():
            pltpu.sync_copy(z1_hbm.at[pl.ds(s * 1000, 1000)], dbuf_v)
            pltpu.sync_copy(dbuf_v, deg_s.at[pl.ds(s * 1000, 1000)])
        for j in range(7):
            ones_v[pl.ds(j * 16, 16)] = jnp.full((16,), 1.0, jnp.float32)
    plsc.subcore_barrier()

    nunits = jnp.where(s < 8, 13, 12)
    unit0 = c * (UNITS // NC) + 12 * s + jnp.minimum(s, 8)
    bufs = (rows_a, rows_b, rows_c)
    gsems = (sem_a, sem_b, sem_c)
    ssems = (sem_d, sem_e, sem_f)
    NB = len(bufs)

    def fetch_idx(u, slot):
        irow = pl.multiple_of((unit0 + u) * UR, 8)
        pltpu.async_copy(ei_hbm.at[0, pl.ds(irow, UR)], src_v.at[slot],
                         isem_s)
        pltpu.async_copy(ei_hbm.at[1, pl.ds(irow, UR)], dst_v.at[slot],
                         isem_d)

    fetch_idx(0, 0)
    pltpu.make_async_copy(ei_hbm.at[0, pl.ds(0, UR)], src_v.at[0],
                          isem_s).wait()
    pltpu.make_async_copy(ei_hbm.at[1, pl.ds(0, UR)], dst_v.at[0],
                          isem_d).wait()

    def unit_body(u, carry):
        @pl.when(u < nunits)
        def _():
            p = u % 2

            @pl.when(u + 1 < nunits)
            def _():
                fetch_idx(u + 1, 1 - p)

            @pl.when(u > 0)
            def _():
                pltpu.make_async_copy(ei_hbm.at[0, pl.ds(0, UR)],
                                      src_v.at[p], isem_s).wait()
                pltpu.make_async_copy(ei_hbm.at[1, pl.ds(0, UR)],
                                      dst_v.at[p], isem_d).wait()

            inflight = []
            pend_sc = [None] * NB

            def scatter_oldest():
                pd, pb, pj = inflight.pop(0)
                pd.wait()
                pend_sc[pb] = pltpu.async_copy(
                    bufs[pb], acc_s.at[dst_v.at[p, pj]], ssems[pb],
                    add=True)
                if with_deg:
                    pltpu.sync_copy(ones_v.at[pl.ds(0, IR)],
                                    deg_s.at[dst_v.at[p, pj]], add=True)

            for j in range(UR):
                b = j % NB
                if pend_sc[b] is not None:
                    pend_sc[b].wait()
                    pend_sc[b] = None
                inflight.append(
                    (pltpu.async_copy(h_hbm.at[src_v.at[p, j]], bufs[b],
                                      gsems[b]), b, j))
                if len(inflight) == NB:
                    scatter_oldest()
            while inflight:
                scatter_oldest()
            for dd in pend_sc:
                if dd is not None:
                    dd.wait()
        return carry

    lax.fori_loop(0, 13, unit_body, 0)
    plsc.subcore_barrier()

    @pl.when(s < NS - 1)
    def _():
        bounce_rows(lambda r, n: acc_s.at[pl.ds(r, n)],
                    lambda r, n: agg_hbm.at[c, pl.ds(r, n)], row0, RSPLIT)

    @pl.when(s == NS - 1)
    def _():
        bounce_rows(lambda r, n: acc_s.at[pl.ds(r, n)],
                    lambda r, n: agg_hbm.at[c, pl.ds(r, n)],
                    (NS - 1) * RSPLIT, RLAST)

    if with_deg:
        @pl.when(s < 10)
        def _():
            pltpu.sync_copy(deg_s.at[pl.ds(s * 1000, 1000)], dbuf_v)

        @pl.when(jnp.logical_and(c == 0, s < 10))
        def _():
            pltpu.sync_copy(dbuf_v, deg0_hbm.at[pl.ds(s * 1000, 1000)])

        @pl.when(jnp.logical_and(c == 1, s < 10))
        def _():
            pltpu.sync_copy(dbuf_v, deg1_hbm.at[pl.ds(s * 1000, 1000)])


def _make_seg(with_deg):
    return pl.kernel(
        functools.partial(_seg_body, with_deg),
        out_type=(jax.ShapeDtypeStruct((NC, N, D), jnp.float32),
                  jax.ShapeDtypeStruct((N,), jnp.float32),
                  jax.ShapeDtypeStruct((N,), jnp.float32)),
        mesh=_mesh,
        scratch_types=[
            pltpu.VMEM((2, UR, IR), jnp.int32),
            pltpu.VMEM((2, UR, IR), jnp.int32),
            pltpu.VMEM((IR, D), jnp.float32),
            pltpu.VMEM((IR, D), jnp.float32),
            pltpu.VMEM((IR, D), jnp.float32),
            pltpu.VMEM((112,), jnp.float32),
            pltpu.VMEM((1000,), jnp.float32),
            pltpu.VMEM_SHARED((N, D), jnp.float32),
            pltpu.VMEM_SHARED((N,), jnp.float32),
            pltpu.SemaphoreType.DMA,
            pltpu.SemaphoreType.DMA,
            pltpu.SemaphoreType.DMA,
            pltpu.SemaphoreType.DMA,
            pltpu.SemaphoreType.DMA,
            pltpu.SemaphoreType.DMA,
            pltpu.SemaphoreType.DMA,
            pltpu.SemaphoreType.DMA,
        ],
    )


_seg_deg = _make_seg(True)
_seg_nodeg = _make_seg(False)

MB = 2000


def _self_body(x_ref, ws_ref, b_ref, o_ref):
    o_ref[...] = (jnp.dot(x_ref[...], ws_ref[...],
                          preferred_element_type=jnp.float32) + b_ref[...])


def _comb1_body(xs_ref, a_ref, d_ref, wn_ref, o_ref):
    d = d_ref[...]
    rdeg = 1.0 / jnp.maximum(d[:, 0:1] + d[:, 1:2], 1.0)
    a = a_ref[...]
    agg = (a[0] + a[1]) * rdeg
    o_ref[...] = jnp.maximum(
        xs_ref[...]
        + jnp.dot(agg, wn_ref[...], preferred_element_type=jnp.float32), 0.0)


def _comb2_body(hs_ref, a_ref, d_ref, wn_ref, wo_ref, bo_ref, o_ref):
    d = d_ref[...]
    rdeg = 1.0 / jnp.maximum(d[:, 0:1] + d[:, 1:2], 1.0)
    a = a_ref[...]
    agg = (a[0] + a[1]) * rdeg
    h2 = jnp.maximum(
        hs_ref[...]
        + jnp.dot(agg, wn_ref[...], preferred_element_type=jnp.float32), 0.0)
    o_ref[...] = (jnp.dot(h2, wo_ref[...], preferred_element_type=jnp.float32)
                  + bo_ref[...])


def _w_spec(shape):
    return pl.BlockSpec(shape, lambda m: (0,) * len(shape))


_row_spec = pl.BlockSpec((MB, D), lambda m: (m, 0))
_agg_spec = pl.BlockSpec((NC, MB, D), lambda m: (0, m, 0))
_deg_spec = pl.BlockSpec((MB, NC), lambda m: (m, 0))
_params = pltpu.CompilerParams(dimension_semantics=("parallel",))

_self_mm = pl.pallas_call(
    _self_body,
    grid=(N // MB,),
    in_specs=[_row_spec, _w_spec((D, D)), _w_spec((1, D))],
    out_specs=_row_spec,
    out_shape=jax.ShapeDtypeStruct((N, D), jnp.float32),
    compiler_params=_params,
)

_comb1 = pl.pallas_call(
    _comb1_body,
    grid=(N // MB,),
    in_specs=[_row_spec, _agg_spec, _deg_spec, _w_spec((D, D))],
    out_specs=_row_spec,
    out_shape=jax.ShapeDtypeStruct((N, D), jnp.float32),
    compiler_params=_params,
)

_comb2 = pl.pallas_call(
    _comb2_body,
    grid=(N // MB,),
    in_specs=[_row_spec, _agg_spec, _deg_spec, _w_spec((D, D)),
              _w_spec((D, OUTP)), _w_spec((1, OUTP))],
    out_specs=pl.BlockSpec((MB, OUTP), lambda m: (m, 0)),
    out_shape=jax.ShapeDtypeStruct((N, OUTP), jnp.float32),
    compiler_params=_params,
)


def kernel(x, edge_index, W1_self, W1_nbr, b1, W2_self, W2_nbr, b2, W_out,
           b_out):
    ei = edge_index.reshape(2, E // IR, IR)
    z2 = jnp.zeros((N, D), jnp.float32)
    z1 = jnp.zeros((N,), jnp.float32)

    agg_x, deg0, deg1 = _seg_deg(x, ei, z2, z1)
    xs = _self_mm(x, W1_self, b1.reshape(1, D))
    degT = jnp.stack([deg0, deg1], axis=1)
    h1 = _comb1(xs, agg_x, degT, W1_nbr)
    agg_h1, _, _ = _seg_nodeg(h1, ei, z2, z1)
    hs = _self_mm(h1, W2_self, b2.reshape(1, D))
    return _comb2(hs, agg_h1, degT, W2_nbr, W_out, b_out.reshape(1, OUTP))


# speedup vs baseline: 1.0775x; 1.0775x over previous
import functools

import jax
import jax.numpy as jnp
from jax import lax
from jax.experimental import pallas as pl
from jax.experimental.pallas import tpu as pltpu
from jax.experimental.pallas import tpu_sc as plsc

N = 10000
D = 128
E = 320000
OUTP = 3

NC = 2
NS = 16
NW = NC * NS
IR = 100
UR = 16
UNITS = E // (IR * UR)
UT = 8
CB = 96
RSPLIT = 632
RLAST = N - 15 * RSPLIT

_mesh = plsc.VectorSubcoreMesh(core_axis_name="c", subcore_axis_name="s")


def _seg_body(with_deg, h_hbm, ei_hbm, z2_hbm, z1_hbm,
              agg_hbm, deg0_hbm, deg1_hbm,
              src_v, dst_v, rows_a, rows_b, rows_c, ones_v, dbuf_v,
              acc_s, deg_s,
              sem_a, sem_b, sem_c, sem_d, sem_e, sem_f, isem_s, isem_d):
    c = lax.axis_index("c")
    s = lax.axis_index("s")
    w = c * NS + s

    def bounce_rows(src_at, dst_at, row0, nrows):
        full, rem = nrows // CB, nrows % CB
        for i in range(full):
            r = pl.multiple_of(row0 + i * CB, 8)
            pltpu.sync_copy(src_at(r, CB), rows_a.at[pl.ds(0, CB)])
            pltpu.sync_copy(rows_a.at[pl.ds(0, CB)], dst_at(r, CB))
        if rem:
            r = pl.multiple_of(row0 + full * CB, 8)
            pltpu.sync_copy(src_at(r, rem), rows_a.at[pl.ds(0, rem)])
            pltpu.sync_copy(rows_a.at[pl.ds(0, rem)], dst_at(r, rem))

    row0 = pl.multiple_of(s * RSPLIT, 8)

    @pl.when(s < NS - 1)
    def _():
        bounce_rows(lambda r, n: z2_hbm.at[pl.ds(r, n)],
                    lambda r, n: acc_s.at[pl.ds(r, n)], row0, RSPLIT)

    @pl.when(s == NS - 1)
    def _():
        bounce_rows(lambda r, n: z2_hbm.at[pl.ds(r, n)],
                    lambda r, n: acc_s.at[pl.ds(r, n)],
                    (NS - 1) * RSPLIT, RLAST)

    if with_deg:
        @pl.when(s < 10)
        def _():
            pltpu.sync_copy(z1_hbm.at[pl.ds(s * 1000, 1000)], dbuf_v)
            pltpu.sync_copy(dbuf_v, deg_s.at[pl.ds(s * 1000, 1000)])
        for j in range(7):
            ones_v[pl.ds(j * 16, 16)] = jnp.full((16,), 1.0, jnp.float32)
    plsc.subcore_barrier()

    rstart = c * (E // IR // NC) + (UR * 6) * s + UT * jnp.minimum(s, 8)
    bufs = (rows_a, rows_b, rows_c)
    gsems = (sem_a, sem_b, sem_c)
    ssems = (sem_d, sem_e, sem_f)
    NB = len(bufs)

    def fetch_idx(u, slot, nrows=UR):
        irow = pl.multiple_of(rstart + u * UR, 8)
        pltpu.async_copy(ei_hbm.at[0, pl.ds(irow, nrows)],
                         src_v.at[slot, pl.ds(0, nrows)], isem_s)
        pltpu.async_copy(ei_hbm.at[1, pl.ds(irow, nrows)],
                         dst_v.at[slot, pl.ds(0, nrows)], isem_d)

    def drain_idx(slot, nrows=UR):
        pltpu.make_async_copy(ei_hbm.at[0, pl.ds(0, nrows)],
                              src_v.at[slot, pl.ds(0, nrows)],
                              isem_s).wait()
        pltpu.make_async_copy(ei_hbm.at[1, pl.ds(0, nrows)],
                              dst_v.at[slot, pl.ds(0, nrows)],
                              isem_d).wait()

    def run_unit(p, nstreams, with_deg_unit=with_deg):
        inflight = []
        pend_sc = [None] * NB

        def scatter_oldest():
            pd, pb, pj = inflight.pop(0)
            pd.wait()
            pend_sc[pb] = pltpu.async_copy(
                bufs[pb], acc_s.at[dst_v.at[p, pj]], ssems[pb], add=True)
            if with_deg_unit:
                pltpu.sync_copy(ones_v.at[pl.ds(0, IR)],
                                deg_s.at[dst_v.at[p, pj]], add=True)

        for j in range(nstreams):
            b = j % NB
            if pend_sc[b] is not None:
                pend_sc[b].wait()
                pend_sc[b] = None
            inflight.append(
                (pltpu.async_copy(h_hbm.at[src_v.at[p, j]], bufs[b],
                                  gsems[b]), b, j))
            if len(inflight) == NB:
                scatter_oldest()
        while inflight:
            scatter_oldest()
        for dd in pend_sc:
            if dd is not None:
                dd.wait()

    fetch_idx(0, 0)
    drain_idx(0)

    def unit_body(u, carry):
        p = u % 2

        @pl.when(u + 1 < 6)
        def _():
            fetch_idx(u + 1, 1 - p)

        @pl.when(jnp.logical_and(u == 5, s < 8))
        def _():
            fetch_idx(6, 1 - p, UT)

        @pl.when(u > 0)
        def _():
            drain_idx(p)

        run_unit(p, UR)
        return carry

    lax.fori_loop(0, 6, unit_body, 0)

    @pl.when(s < 8)
    def _():
        drain_idx(0, UT)
        run_unit(0, UT)
    plsc.subcore_barrier()

    @pl.when(s < NS - 1)
    def _():
        bounce_rows(lambda r, n: acc_s.at[pl.ds(r, n)],
                    lambda r, n: agg_hbm.at[c, pl.ds(r, n)], row0, RSPLIT)

    @pl.when(s == NS - 1)
    def _():
        bounce_rows(lambda r, n: acc_s.at[pl.ds(r, n)],
                    lambda r, n: agg_hbm.at[c, pl.ds(r, n)],
                    (NS - 1) * RSPLIT, RLAST)

    if with_deg:
        @pl.when(s < 10)
        def _():
            pltpu.sync_copy(deg_s.at[pl.ds(s * 1000, 1000)], dbuf_v)

        @pl.when(jnp.logical_and(c == 0, s < 10))
        def _():
            pltpu.sync_copy(dbuf_v, deg0_hbm.at[pl.ds(s * 1000, 1000)])

        @pl.when(jnp.logical_and(c == 1, s < 10))
        def _():
            pltpu.sync_copy(dbuf_v, deg1_hbm.at[pl.ds(s * 1000, 1000)])


def _make_seg(with_deg):
    return pl.kernel(
        functools.partial(_seg_body, with_deg),
        out_type=(jax.ShapeDtypeStruct((NC, N, D), jnp.float32),
                  jax.ShapeDtypeStruct((N,), jnp.float32),
                  jax.ShapeDtypeStruct((N,), jnp.float32)),
        mesh=_mesh,
        scratch_types=[
            pltpu.VMEM((2, UR, IR), jnp.int32),
            pltpu.VMEM((2, UR, IR), jnp.int32),
            pltpu.VMEM((IR, D), jnp.float32),
            pltpu.VMEM((IR, D), jnp.float32),
            pltpu.VMEM((IR, D), jnp.float32),
            pltpu.VMEM((112,), jnp.float32),
            pltpu.VMEM((1000,), jnp.float32),
            pltpu.VMEM_SHARED((N, D), jnp.float32),
            pltpu.VMEM_SHARED((N,), jnp.float32),
            pltpu.SemaphoreType.DMA,
            pltpu.SemaphoreType.DMA,
            pltpu.SemaphoreType.DMA,
            pltpu.SemaphoreType.DMA,
            pltpu.SemaphoreType.DMA,
            pltpu.SemaphoreType.DMA,
            pltpu.SemaphoreType.DMA,
            pltpu.SemaphoreType.DMA,
        ],
    )


_seg_deg = _make_seg(True)
_seg_nodeg = _make_seg(False)

MB = 2000


def _self_body(x_ref, ws_ref, b_ref, o_ref):
    o_ref[...] = (jnp.dot(x_ref[...], ws_ref[...],
                          preferred_element_type=jnp.float32) + b_ref[...])


def _comb1_body(xs_ref, a_ref, d_ref, wn_ref, o_ref):
    d = d_ref[...]
    rdeg = 1.0 / jnp.maximum(d[:, 0:1] + d[:, 1:2], 1.0)
    a = a_ref[...]
    agg = (a[0] + a[1]) * rdeg
    o_ref[...] = jnp.maximum(
        xs_ref[...]
        + jnp.dot(agg, wn_ref[...], preferred_element_type=jnp.float32), 0.0)


def _comb2_body(hs_ref, a_ref, d_ref, wn_ref, wo_ref, bo_ref, o_ref):
    d = d_ref[...]
    rdeg = 1.0 / jnp.maximum(d[:, 0:1] + d[:, 1:2], 1.0)
    a = a_ref[...]
    agg = (a[0] + a[1]) * rdeg
    h2 = jnp.maximum(
        hs_ref[...]
        + jnp.dot(agg, wn_ref[...], preferred_element_type=jnp.float32), 0.0)
    o_ref[...] = (jnp.dot(h2, wo_ref[...], preferred_element_type=jnp.float32)
                  + bo_ref[...])


def _w_spec(shape):
    return pl.BlockSpec(shape, lambda m: (0,) * len(shape))


_row_spec = pl.BlockSpec((MB, D), lambda m: (m, 0))
_agg_spec = pl.BlockSpec((NC, MB, D), lambda m: (0, m, 0))
_deg_spec = pl.BlockSpec((MB, NC), lambda m: (m, 0))
_params = pltpu.CompilerParams(dimension_semantics=("parallel",))

_self_mm = pl.pallas_call(
    _self_body,
    grid=(N // MB,),
    in_specs=[_row_spec, _w_spec((D, D)), _w_spec((1, D))],
    out_specs=_row_spec,
    out_shape=jax.ShapeDtypeStruct((N, D), jnp.float32),
    compiler_params=_params,
)

_comb1 = pl.pallas_call(
    _comb1_body,
    grid=(N // MB,),
    in_specs=[_row_spec, _agg_spec, _deg_spec, _w_spec((D, D))],
    out_specs=_row_spec,
    out_shape=jax.ShapeDtypeStruct((N, D), jnp.float32),
    compiler_params=_params,
)

_comb2 = pl.pallas_call(
    _comb2_body,
    grid=(N // MB,),
    in_specs=[_row_spec, _agg_spec, _deg_spec, _w_spec((D, D)),
              _w_spec((D, OUTP)), _w_spec((1, OUTP))],
    out_specs=pl.BlockSpec((MB, OUTP), lambda m: (m, 0)),
    out_shape=jax.ShapeDtypeStruct((N, OUTP), jnp.float32),
    compiler_params=_params,
)


def kernel(x, edge_index, W1_self, W1_nbr, b1, W2_self, W2_nbr, b2, W_out,
           b_out):
    ei = edge_index.reshape(2, E // IR, IR)
    z2 = jnp.zeros((N, D), jnp.float32)
    z1 = jnp.zeros((N,), jnp.float32)

    agg_x, deg0, deg1 = _seg_deg(x, ei, z2, z1)
    xs = _self_mm(x, W1_self, b1.reshape(1, D))
    degT = jnp.stack([deg0, deg1], axis=1)
    h1 = _comb1(xs, agg_x, degT, W1_nbr)
    agg_h1, _, _ = _seg_nodeg(h1, ei, z2, z1)
    hs = _self_mm(h1, W2_self, b2.reshape(1, D))
    return _comb2(hs, agg_h1, degT, W2_nbr, W_out, b_out.reshape(1, OUTP))
